# Initial kernel scaffold; baseline (speedup 1.0000x reference)
import jax
import jax.numpy as jnp
from jax.experimental import pallas as pl


def kernel(atom_features_list, bond_info, bn_gamma1, bn_beta1, W1, bn_gamma2, bn_beta2, W2):
    raise NotImplementedError("write your pallas kernel here")


# speedup vs baseline: 2.0217x; 2.0217x over previous
import functools

import jax
import jax.numpy as jnp
from jax import lax
from jax.experimental import pallas as pl
from jax.experimental.pallas import tpu as pltpu
from jax.experimental.pallas import tpu_sc as plsc

N = 10000
E = 320000
NBT = 4
F = 128
CW = 32
NSUB = 16
EPT = 20480
EPAD = NSUB * EPT
ROWS_B = EPT // 128
HP = 10240
PR = 10400
ACC_R = NBT * PR
DUMMY = N
EB = 2000
RB = 400
IGRP = 8
NGRP = ROWS_B // IGRP


PKSHIFT = 14


def _edge_body(bond_ref, pk_ref):
    blk = bond_ref[...]
    dst = (blk[:, 2] & (NBT - 1)) * PR + blk[:, 1]
    pk_ref[0, 0, :] = dst * (1 << PKSHIFT) + blk[:, 0]


def _edge_indices(bond_info):
    pkf = pl.pallas_call(
        _edge_body,
        grid=(E // EB,),
        in_specs=[pl.BlockSpec((EB, 3), lambda i: (i, 0))],
        out_specs=pl.BlockSpec((1, 1, EB), lambda i: (i, 0, 0)),
        out_shape=jax.ShapeDtypeStruct((E // EB, 1, EB), jnp.int32),
    )(bond_info)
    pad = jnp.full((EPAD - E,), DUMMY * (1 << PKSHIFT), jnp.int32)
    pk = jnp.concatenate([pkf.reshape(E), pad])
    return pk.reshape(NSUB, ROWS_B, 128)


def _stats1_body(af_ref, o_ref):
    x = jnp.concatenate([af_ref[0], af_ref[1]], axis=-1)
    upd = jnp.concatenate(
        [jnp.sum(x, axis=0, keepdims=True),
         jnp.sum(x * x, axis=0, keepdims=True),
         jnp.zeros((6, F), jnp.float32)], axis=0)

    @pl.when(pl.program_id(0) == 0)
    def _():
        o_ref[...] = jnp.zeros_like(o_ref)

    o_ref[...] += upd


def _stats1(af):
    return pl.pallas_call(
        _stats1_body,
        grid=(N // RB,),
        in_specs=[pl.BlockSpec((2, RB, 64), lambda i: (0, i, 0))],
        out_specs=pl.BlockSpec((8, F), lambda i: (0, 0)),
        out_shape=jax.ShapeDtypeStruct((8, F), jnp.float32),
    )(af)


def _bn_elu(x, s1, s2, g, b):
    m = s1 / N
    v = s2 / N - m * m
    xn = (x - m) * lax.rsqrt(v + 1e-5) * g + b
    return jnp.where(xn > 0, xn, jnp.exp(xn) - 1.0)


def _h_body(af_ref, st_ref, g_ref, b_ref, w_ref,
            h_ref, c0_ref, c1_ref, c2_ref, c3_ref, hs_ref):
    x = jnp.concatenate([af_ref[0], af_ref[1]], axis=-1)
    a = _bn_elu(x, st_ref[0:1, :], st_ref[1:2, :], g_ref[...], b_ref[...])
    h = jnp.dot(a, w_ref[...], preferred_element_type=jnp.float32)
    h_ref[...] = h
    c0_ref[...] = h[:, 0 * CW:1 * CW]
    c1_ref[...] = h[:, 1 * CW:2 * CW]
    c2_ref[...] = h[:, 2 * CW:3 * CW]
    c3_ref[...] = h[:, 3 * CW:4 * CW]
    upd = jnp.concatenate(
        [jnp.sum(h, axis=0, keepdims=True),
         jnp.sum(h * h, axis=0, keepdims=True),
         jnp.zeros((6, F), jnp.float32)], axis=0)

    @pl.when(pl.program_id(0) == 0)
    def _():
        hs_ref[...] = jnp.zeros_like(hs_ref)

    hs_ref[...] += upd


def _bottleneck(af, stats1, g1, b1, W1):
    chunk_spec = pl.BlockSpec((RB, CW), lambda i: (i, 0))
    return pl.pallas_call(
        _h_body,
        grid=(N // RB,),
        in_specs=[
            pl.BlockSpec((2, RB, 64), lambda i: (0, i, 0)),
            pl.BlockSpec((8, F), lambda i: (0, 0)),
            pl.BlockSpec((1, F), lambda i: (0, 0)),
            pl.BlockSpec((1, F), lambda i: (0, 0)),
            pl.BlockSpec((F, F), lambda i: (0, 0)),
        ],
        out_specs=[
            pl.BlockSpec((RB, F), lambda i: (i, 0)),
            chunk_spec, chunk_spec, chunk_spec, chunk_spec,
            pl.BlockSpec((8, F), lambda i: (0, 0)),
        ],
        out_shape=[
            jax.ShapeDtypeStruct((HP, F), jnp.float32),
            jax.ShapeDtypeStruct((HP, CW), jnp.float32),
            jax.ShapeDtypeStruct((HP, CW), jnp.float32),
            jax.ShapeDtypeStruct((HP, CW), jnp.float32),
            jax.ShapeDtypeStruct((HP, CW), jnp.float32),
            jax.ShapeDtypeStruct((8, F), jnp.float32),
        ],
    )(af, stats1, g1.reshape(1, F), b1.reshape(1, F), W1)


_SC_MESH = plsc.VectorSubcoreMesh(core_axis_name="c", subcore_axis_name="s")

_ZSTRIPE = ACC_R // NSUB
_HSTRIPE = HP // NSUB


@functools.partial(
    pl.kernel,
    mesh=_SC_MESH,
    compiler_params=pltpu.CompilerParams(use_tc_tiling_on_sc=False),
    out_type=[jax.ShapeDtypeStruct((ACC_R, CW), jnp.float32)] * 4,
    scratch_types=[
        pltpu.VMEM((IGRP, 128), jnp.int32),
        pltpu.VMEM((IGRP, 128), jnp.int32),
        pltpu.VMEM((IGRP, 128), jnp.int32),
        pltpu.VMEM((128, CW), jnp.float32),
        pltpu.VMEM((128, CW), jnp.float32),
        pltpu.VMEM_SHARED((ACC_R, CW), jnp.float32),
    ],
)
def _molconv_sc(h0, h1, h2, h3, pk_hbm, o0, o1, o2, o3,
                pk_v, beg_v, dst_v, rows_v, zero_v, acc_sh):
    c = lax.axis_index("c")
    s = lax.axis_index("s")

    @pl.loop(0, 128)
    def _(i):
        @pl.loop(0, CW, step=16)
        def _(k):
            zero_v[i, pl.ds(k, 16)] = jnp.zeros((16,), jnp.float32)

    def chunk_pass(h_chunk_hbm, out_hbm):
        zbase = s * _ZSTRIPE
        for q in range(_ZSTRIPE // 128):
            pltpu.sync_copy(zero_v, acc_sh.at[pl.ds(zbase + q * 128, 128)])
        rem = _ZSTRIPE % 128
        if rem:
            pltpu.sync_copy(zero_v.at[pl.ds(0, rem)],
                            acc_sh.at[pl.ds(zbase + _ZSTRIPE - rem, rem)])
        plsc.subcore_barrier()

        @pl.loop(0, NGRP)
        def _(g):
            pltpu.sync_copy(pk_hbm.at[s, pl.ds(g * IGRP, IGRP)], pk_v)

            @pl.loop(0, IGRP)
            def _(r):
                @pl.loop(0, 128, step=16)
                def _(k):
                    w = pk_v[r, pl.ds(k, 16)]
                    beg_v[r, pl.ds(k, 16)] = w & ((1 << PKSHIFT) - 1)
                    dst_v[r, pl.ds(k, 16)] = lax.shift_right_logical(w, PKSHIFT)

            @pl.loop(0, IGRP)
            def _(bb):
                pltpu.sync_copy(h_chunk_hbm.at[beg_v.at[bb]], rows_v)
                pltpu.sync_copy(rows_v, acc_sh.at[dst_v.at[bb]], add=True)

        plsc.subcore_barrier()
        pltpu.sync_copy(acc_sh.at[pl.ds(s * _ZSTRIPE, _ZSTRIPE)],
                        out_hbm.at[pl.ds(s * _ZSTRIPE, _ZSTRIPE)])
        plsc.subcore_barrier()

    for j in range(2):
        @pl.when(c == 0)
        def _(j=j):
            chunk_pass((h0, h1)[j], (o0, o1)[j])

        @pl.when(c == 1)
        def _(j=j):
            chunk_pass((h2, h3)[j], (o2, o3)[j])


def _panel_specs():
    return [pl.BlockSpec((RB, CW), (lambda i, t=t: (t * (PR // RB) + i, 0)))
            for t in range(NBT) for _ in range(4)]


def _stats2_body(*refs):
    panel_refs, o_ref = refs[:-1], refs[-1]
    s1, s2 = [], []
    for t in range(NBT):
        x = jnp.concatenate([panel_refs[4 * t + cc][...] for cc in range(4)],
                            axis=-1)
        s1.append(jnp.sum(x, axis=0, keepdims=True))
        s2.append(jnp.sum(x * x, axis=0, keepdims=True))
    upd = jnp.concatenate(
        [jnp.concatenate(s1, axis=-1),
         jnp.concatenate(s2, axis=-1),
         jnp.zeros((6, NBT * F), jnp.float32)], axis=0)

    @pl.when(pl.program_id(0) == 0)
    def _():
        o_ref[...] = jnp.zeros_like(o_ref)

    o_ref[...] += upd


def _stats2(bufs):
    return pl.pallas_call(
        _stats2_body,
        grid=(N // RB,),
        in_specs=_panel_specs(),
        out_specs=pl.BlockSpec((8, NBT * F), lambda i: (0, 0)),
        out_shape=jax.ShapeDtypeStruct((8, NBT * F), jnp.float32),
    )(*(bufs * NBT))


def _out_body(*refs):
    h_ref = refs[0]
    panel_refs = refs[1:17]
    hs_ref, bs_ref, g_ref, b_ref, w_ref, o_ref = refs[17:]
    a = _bn_elu(h_ref[...], hs_ref[0:1, :], hs_ref[1:2, :],
                g_ref[0:1, 0:F], b_ref[0:1, 0:F])
    acc = jnp.dot(a, w_ref[0:F, :], preferred_element_type=jnp.float32)
    for t in range(NBT):
        x = jnp.concatenate([panel_refs[4 * t + cc][...] for cc in range(4)],
                            axis=-1)
        c0 = (t + 1) * F
        at = _bn_elu(x, bs_ref[0:1, t * F:(t + 1) * F],
                     bs_ref[1:2, t * F:(t + 1) * F],
                     g_ref[0:1, c0:c0 + F], b_ref[0:1, c0:c0 + F])
        acc += jnp.dot(at, w_ref[c0:c0 + F, :],
                       preferred_element_type=jnp.float32)
    o_ref[...] = acc


def _head(h, bufs, hstats, bstats, g2, b2, W2):
    cd = (NBT + 1) * F
    return pl.pallas_call(
        _out_body,
        grid=(N // RB,),
        in_specs=[pl.BlockSpec((RB, F), lambda i: (i, 0))] + _panel_specs() + [
            pl.BlockSpec((8, F), lambda i: (0, 0)),
            pl.BlockSpec((8, NBT * F), lambda i: (0, 0)),
            pl.BlockSpec((1, cd), lambda i: (0, 0)),
            pl.BlockSpec((1, cd), lambda i: (0, 0)),
            pl.BlockSpec((cd, F), lambda i: (0, 0)),
        ],
        out_specs=pl.BlockSpec((RB, F), lambda i: (i, 0)),
        out_shape=jax.ShapeDtypeStruct((N, F), jnp.float32),
    )(h, *(bufs * NBT), hstats, bstats,
      g2.reshape(1, cd), b2.reshape(1, cd), W2)


def kernel(atom_features_list, bond_info, bn_gamma1, bn_beta1, W1,
           bn_gamma2, bn_beta2, W2):
    af = atom_features_list
    pk = _edge_indices(bond_info)
    stats1 = _stats1(af)
    h, h0, h1, h2, h3, hstats = _bottleneck(af, stats1, bn_gamma1, bn_beta1, W1)
    bufs = list(_molconv_sc(h0, h1, h2, h3, pk))
    bstats = _stats2(bufs)
    return _head(h, bufs, hstats, bstats, bn_gamma2, bn_beta2, W2)


# speedup vs baseline: 2.2710x; 1.1233x over previous
import functools

import jax
import jax.numpy as jnp
from jax import lax
from jax.experimental import pallas as pl
from jax.experimental.pallas import tpu as pltpu
from jax.experimental.pallas import tpu_sc as plsc

N = 10000
E = 320000
NBT = 4
F = 128
CW = 32
NSUB = 16
EPT = 20480
EPAD = NSUB * EPT
ROWS_B = EPT // 128
HP = 10240
PR = 10400
ACC_R = NBT * PR
DUMMY = N
EB = 2000
RB = 400
IGRP = 8
NGRP = ROWS_B // IGRP


PKSHIFT = 14


def _edge_body(bond_ref, pk_ref):
    blk = bond_ref[...]
    dst = (blk[:, 2] & (NBT - 1)) * PR + blk[:, 1]
    pk_ref[0, 0, :] = dst * (1 << PKSHIFT) + blk[:, 0]


def _edge_indices(bond_info):
    pkf = pl.pallas_call(
        _edge_body,
        grid=(E // EB,),
        in_specs=[pl.BlockSpec((EB, 3), lambda i: (i, 0))],
        out_specs=pl.BlockSpec((1, 1, EB), lambda i: (i, 0, 0)),
        out_shape=jax.ShapeDtypeStruct((E // EB, 1, EB), jnp.int32),
    )(bond_info)
    pad = jnp.full((EPAD - E,), DUMMY * (1 << PKSHIFT), jnp.int32)
    pk = jnp.concatenate([pkf.reshape(E), pad])
    return pk.reshape(NSUB, ROWS_B, 128)


def _stats1_body(af_ref, o_ref):
    x = jnp.concatenate([af_ref[0], af_ref[1]], axis=-1)
    upd = jnp.concatenate(
        [jnp.sum(x, axis=0, keepdims=True),
         jnp.sum(x * x, axis=0, keepdims=True),
         jnp.zeros((6, F), jnp.float32)], axis=0)

    @pl.when(pl.program_id(0) == 0)
    def _():
        o_ref[...] = jnp.zeros_like(o_ref)

    o_ref[...] += upd


def _stats1(af):
    return pl.pallas_call(
        _stats1_body,
        grid=(N // RB,),
        in_specs=[pl.BlockSpec((2, RB, 64), lambda i: (0, i, 0))],
        out_specs=pl.BlockSpec((8, F), lambda i: (0, 0)),
        out_shape=jax.ShapeDtypeStruct((8, F), jnp.float32),
    )(af)


def _bn_elu(x, s1, s2, g, b):
    m = s1 / N
    v = s2 / N - m * m
    xn = (x - m) * lax.rsqrt(v + 1e-5) * g + b
    return jnp.where(xn > 0, xn, jnp.exp(xn) - 1.0)


def _h_body(af_ref, st_ref, g_ref, b_ref, w_ref,
            h_ref, c0_ref, c1_ref, c2_ref, c3_ref, hs_ref):
    x = jnp.concatenate([af_ref[0], af_ref[1]], axis=-1)
    a = _bn_elu(x, st_ref[0:1, :], st_ref[1:2, :], g_ref[...], b_ref[...])
    h = jnp.dot(a, w_ref[...], preferred_element_type=jnp.float32)
    h_ref[...] = h
    c0_ref[...] = h[:, 0 * CW:1 * CW]
    c1_ref[...] = h[:, 1 * CW:2 * CW]
    c2_ref[...] = h[:, 2 * CW:3 * CW]
    c3_ref[...] = h[:, 3 * CW:4 * CW]
    upd = jnp.concatenate(
        [jnp.sum(h, axis=0, keepdims=True),
         jnp.sum(h * h, axis=0, keepdims=True),
         jnp.zeros((6, F), jnp.float32)], axis=0)

    @pl.when(pl.program_id(0) == 0)
    def _():
        hs_ref[...] = jnp.zeros_like(hs_ref)

    hs_ref[...] += upd


def _bottleneck(af, stats1, g1, b1, W1):
    chunk_spec = pl.BlockSpec((RB, CW), lambda i: (i, 0))
    return pl.pallas_call(
        _h_body,
        grid=(N // RB,),
        in_specs=[
            pl.BlockSpec((2, RB, 64), lambda i: (0, i, 0)),
            pl.BlockSpec((8, F), lambda i: (0, 0)),
            pl.BlockSpec((1, F), lambda i: (0, 0)),
            pl.BlockSpec((1, F), lambda i: (0, 0)),
            pl.BlockSpec((F, F), lambda i: (0, 0)),
        ],
        out_specs=[
            pl.BlockSpec((RB, F), lambda i: (i, 0)),
            chunk_spec, chunk_spec, chunk_spec, chunk_spec,
            pl.BlockSpec((8, F), lambda i: (0, 0)),
        ],
        out_shape=[
            jax.ShapeDtypeStruct((HP, F), jnp.float32),
            jax.ShapeDtypeStruct((HP, CW), jnp.float32),
            jax.ShapeDtypeStruct((HP, CW), jnp.float32),
            jax.ShapeDtypeStruct((HP, CW), jnp.float32),
            jax.ShapeDtypeStruct((HP, CW), jnp.float32),
            jax.ShapeDtypeStruct((8, F), jnp.float32),
        ],
    )(af, stats1, g1.reshape(1, F), b1.reshape(1, F), W1)


_SC_MESH = plsc.VectorSubcoreMesh(core_axis_name="c", subcore_axis_name="s")

_ZSTRIPE = ACC_R // NSUB
_HSTRIPE = HP // NSUB


@functools.partial(
    pl.kernel,
    mesh=_SC_MESH,
    compiler_params=pltpu.CompilerParams(use_tc_tiling_on_sc=False),
    out_type=[jax.ShapeDtypeStruct((ACC_R, CW), jnp.float32)] * 4,
    scratch_types=[
        pltpu.VMEM((IGRP, 128), jnp.int32),
        pltpu.VMEM((IGRP, 128), jnp.int32),
        pltpu.VMEM((IGRP, 128), jnp.int32),
        pltpu.VMEM((128, CW), jnp.float32),
        pltpu.VMEM((128, CW), jnp.float32),
        pltpu.VMEM((64, CW), jnp.float32),
        pltpu.VMEM_SHARED((ACC_R, CW), jnp.float32),
        pltpu.SemaphoreType.DMA,
        pltpu.SemaphoreType.DMA,
        pltpu.SemaphoreType.DMA,
        pltpu.SemaphoreType.DMA,
    ],
)
def _molconv_sc(h0, h1, h2, h3, pk_hbm, o0, o1, o2, o3,
                pk_v, beg_v, dst_v, rows_a, rows_b, zero_v, acc_sh,
                gsem_a, gsem_b, ssem_a, ssem_b):
    c = lax.axis_index("c")
    s = lax.axis_index("s")

    @pl.loop(0, 64)
    def _(i):
        @pl.loop(0, CW, step=16)
        def _(k):
            zero_v[i, pl.ds(k, 16)] = jnp.zeros((16,), jnp.float32)

    def chunk_pass(h_chunk_hbm, out_hbm):
        zbase = s * _ZSTRIPE
        for q in range(_ZSTRIPE // 64):
            pltpu.sync_copy(zero_v, acc_sh.at[pl.ds(zbase + q * 64, 64)])
        rem = _ZSTRIPE % 64
        if rem:
            pltpu.sync_copy(zero_v.at[pl.ds(0, rem)],
                            acc_sh.at[pl.ds(zbase + _ZSTRIPE - rem, rem)])
        plsc.subcore_barrier()

        def gat(r, buf, sem):
            return pltpu.async_copy(h_chunk_hbm.at[beg_v.at[r]], buf, sem)

        def sca(r, buf, sem):
            return pltpu.async_copy(buf, acc_sh.at[dst_v.at[r]], sem,
                                    add=True)

        @pl.loop(0, NGRP)
        def _(g):
            pltpu.sync_copy(pk_hbm.at[s, pl.ds(g * IGRP, IGRP)], pk_v)

            @pl.loop(0, IGRP)
            def _(r):
                @pl.loop(0, 128, step=16)
                def _(k):
                    w = pk_v[r, pl.ds(k, 16)]
                    beg_v[r, pl.ds(k, 16)] = w & ((1 << PKSHIFT) - 1)
                    dst_v[r, pl.ds(k, 16)] = lax.shift_right_logical(w, PKSHIFT)

            ga = gat(0, rows_a, gsem_a)
            gb = gat(1, rows_b, gsem_b)
            ga.wait()
            sa = sca(0, rows_a, ssem_a)
            gb.wait()
            sb = sca(1, rows_b, ssem_b)
            for r in range(2, IGRP, 2):
                sa.wait()
                ga = gat(r, rows_a, gsem_a)
                sb.wait()
                gb = gat(r + 1, rows_b, gsem_b)
                ga.wait()
                sa = sca(r, rows_a, ssem_a)
                gb.wait()
                sb = sca(r + 1, rows_b, ssem_b)
            sa.wait()
            sb.wait()

        plsc.subcore_barrier()
        pltpu.sync_copy(acc_sh.at[pl.ds(s * _ZSTRIPE, _ZSTRIPE)],
                        out_hbm.at[pl.ds(s * _ZSTRIPE, _ZSTRIPE)])
        plsc.subcore_barrier()

    for j in range(2):
        @pl.when(c == 0)
        def _(j=j):
            chunk_pass((h0, h1)[j], (o0, o1)[j])

        @pl.when(c == 1)
        def _(j=j):
            chunk_pass((h2, h3)[j], (o2, o3)[j])


def _panel_specs():
    return [pl.BlockSpec((RB, CW), (lambda i, t=t: (t * (PR // RB) + i, 0)))
            for t in range(NBT) for _ in range(4)]


def _stats2_body(*refs):
    panel_refs, o_ref = refs[:-1], refs[-1]
    s1, s2 = [], []
    for t in range(NBT):
        x = jnp.concatenate([panel_refs[4 * t + cc][...] for cc in range(4)],
                            axis=-1)
        s1.append(jnp.sum(x, axis=0, keepdims=True))
        s2.append(jnp.sum(x * x, axis=0, keepdims=True))
    upd = jnp.concatenate(
        [jnp.concatenate(s1, axis=-1),
         jnp.concatenate(s2, axis=-1),
         jnp.zeros((6, NBT * F), jnp.float32)], axis=0)

    @pl.when(pl.program_id(0) == 0)
    def _():
        o_ref[...] = jnp.zeros_like(o_ref)

    o_ref[...] += upd


def _stats2(bufs):
    return pl.pallas_call(
        _stats2_body,
        grid=(N // RB,),
        in_specs=_panel_specs(),
        out_specs=pl.BlockSpec((8, NBT * F), lambda i: (0, 0)),
        out_shape=jax.ShapeDtypeStruct((8, NBT * F), jnp.float32),
    )(*(bufs * NBT))


def _out_body(*refs):
    h_ref = refs[0]
    panel_refs = refs[1:17]
    hs_ref, bs_ref, g_ref, b_ref, w_ref, o_ref = refs[17:]
    a = _bn_elu(h_ref[...], hs_ref[0:1, :], hs_ref[1:2, :],
                g_ref[0:1, 0:F], b_ref[0:1, 0:F])
    acc = jnp.dot(a, w_ref[0:F, :], preferred_element_type=jnp.float32)
    for t in range(NBT):
        x = jnp.concatenate([panel_refs[4 * t + cc][...] for cc in range(4)],
                            axis=-1)
        c0 = (t + 1) * F
        at = _bn_elu(x, bs_ref[0:1, t * F:(t + 1) * F],
                     bs_ref[1:2, t * F:(t + 1) * F],
                     g_ref[0:1, c0:c0 + F], b_ref[0:1, c0:c0 + F])
        acc += jnp.dot(at, w_ref[c0:c0 + F, :],
                       preferred_element_type=jnp.float32)
    o_ref[...] = acc


def _head(h, bufs, hstats, bstats, g2, b2, W2):
    cd = (NBT + 1) * F
    return pl.pallas_call(
        _out_body,
        grid=(N // RB,),
        in_specs=[pl.BlockSpec((RB, F), lambda i: (i, 0))] + _panel_specs() + [
            pl.BlockSpec((8, F), lambda i: (0, 0)),
            pl.BlockSpec((8, NBT * F), lambda i: (0, 0)),
            pl.BlockSpec((1, cd), lambda i: (0, 0)),
            pl.BlockSpec((1, cd), lambda i: (0, 0)),
            pl.BlockSpec((cd, F), lambda i: (0, 0)),
        ],
        out_specs=pl.BlockSpec((RB, F), lambda i: (i, 0)),
        out_shape=jax.ShapeDtypeStruct((N, F), jnp.float32),
    )(h, *(bufs * NBT), hstats, bstats,
      g2.reshape(1, cd), b2.reshape(1, cd), W2)


def kernel(atom_features_list, bond_info, bn_gamma1, bn_beta1, W1,
           bn_gamma2, bn_beta2, W2):
    af = atom_features_list
    pk = _edge_indices(bond_info)
    stats1 = _stats1(af)
    h, h0, h1, h2, h3, hstats = _bottleneck(af, stats1, bn_gamma1, bn_beta1, W1)
    bufs = list(_molconv_sc(h0, h1, h2, h3, pk))
    bstats = _stats2(bufs)
    return _head(h, bufs, hstats, bstats, bn_gamma2, bn_beta2, W2)


# speedup vs baseline: 2.3428x; 1.0316x over previous
import functools

import jax
import jax.numpy as jnp
from jax import lax
from jax.experimental import pallas as pl
from jax.experimental.pallas import tpu as pltpu
from jax.experimental.pallas import tpu_sc as plsc

N = 10000
E = 320000
NBT = 4
F = 128
CW = 32
NSUB = 16
EPT = 20480
EPAD = NSUB * EPT
ROWS_B = EPT // 128
HP = 10240
PR = 10400
ACC_R = NBT * PR
DUMMY = N
EB = 2000
RB = 400
IGRP = 8
NGRP = ROWS_B // IGRP
BS = 256


PKSHIFT = 14


def _edge_body(bond_ref, pk_ref):
    blk = bond_ref[...]
    dst = (blk[:, 2] & (NBT - 1)) * PR + blk[:, 1]
    pk_ref[0, 0, :] = dst * (1 << PKSHIFT) + blk[:, 0]


def _edge_indices(bond_info):
    pkf = pl.pallas_call(
        _edge_body,
        grid=(E // EB,),
        in_specs=[pl.BlockSpec((EB, 3), lambda i: (i, 0))],
        out_specs=pl.BlockSpec((1, 1, EB), lambda i: (i, 0, 0)),
        out_shape=jax.ShapeDtypeStruct((E // EB, 1, EB), jnp.int32),
    )(bond_info)
    pad = jnp.full((EPAD - E,), DUMMY * (1 << PKSHIFT), jnp.int32)
    pk = jnp.concatenate([pkf.reshape(E), pad])
    return pk.reshape(NSUB, ROWS_B, 128)


def _stats1_body(af_ref, o_ref):
    x = jnp.concatenate([af_ref[0], af_ref[1]], axis=-1)
    upd = jnp.concatenate(
        [jnp.sum(x, axis=0, keepdims=True),
         jnp.sum(x * x, axis=0, keepdims=True),
         jnp.zeros((6, F), jnp.float32)], axis=0)

    @pl.when(pl.program_id(0) == 0)
    def _():
        o_ref[...] = jnp.zeros_like(o_ref)

    o_ref[...] += upd


def _stats1(af):
    return pl.pallas_call(
        _stats1_body,
        grid=(N // RB,),
        in_specs=[pl.BlockSpec((2, RB, 64), lambda i: (0, i, 0))],
        out_specs=pl.BlockSpec((8, F), lambda i: (0, 0)),
        out_shape=jax.ShapeDtypeStruct((8, F), jnp.float32),
    )(af)


def _bn_elu(x, s1, s2, g, b):
    m = s1 / N
    v = s2 / N - m * m
    xn = (x - m) * lax.rsqrt(v + 1e-5) * g + b
    return jnp.where(xn > 0, xn, jnp.exp(xn) - 1.0)


def _h_body(af_ref, st_ref, g_ref, b_ref, w_ref,
            h_ref, c0_ref, c1_ref, c2_ref, c3_ref, hs_ref):
    x = jnp.concatenate([af_ref[0], af_ref[1]], axis=-1)
    a = _bn_elu(x, st_ref[0:1, :], st_ref[1:2, :], g_ref[...], b_ref[...])
    h = jnp.dot(a, w_ref[...], preferred_element_type=jnp.float32)
    h_ref[...] = h
    c0_ref[...] = h[:, 0 * CW:1 * CW]
    c1_ref[...] = h[:, 1 * CW:2 * CW]
    c2_ref[...] = h[:, 2 * CW:3 * CW]
    c3_ref[...] = h[:, 3 * CW:4 * CW]
    upd = jnp.concatenate(
        [jnp.sum(h, axis=0, keepdims=True),
         jnp.sum(h * h, axis=0, keepdims=True),
         jnp.zeros((6, F), jnp.float32)], axis=0)

    @pl.when(pl.program_id(0) == 0)
    def _():
        hs_ref[...] = jnp.zeros_like(hs_ref)

    hs_ref[...] += upd


def _bottleneck(af, stats1, g1, b1, W1):
    chunk_spec = pl.BlockSpec((RB, CW), lambda i: (i, 0))
    return pl.pallas_call(
        _h_body,
        grid=(N // RB,),
        in_specs=[
            pl.BlockSpec((2, RB, 64), lambda i: (0, i, 0)),
            pl.BlockSpec((8, F), lambda i: (0, 0)),
            pl.BlockSpec((1, F), lambda i: (0, 0)),
            pl.BlockSpec((1, F), lambda i: (0, 0)),
            pl.BlockSpec((F, F), lambda i: (0, 0)),
        ],
        out_specs=[
            pl.BlockSpec((RB, F), lambda i: (i, 0)),
            chunk_spec, chunk_spec, chunk_spec, chunk_spec,
            pl.BlockSpec((8, F), lambda i: (0, 0)),
        ],
        out_shape=[
            jax.ShapeDtypeStruct((HP, F), jnp.float32),
            jax.ShapeDtypeStruct((HP, CW), jnp.float32),
            jax.ShapeDtypeStruct((HP, CW), jnp.float32),
            jax.ShapeDtypeStruct((HP, CW), jnp.float32),
            jax.ShapeDtypeStruct((HP, CW), jnp.float32),
            jax.ShapeDtypeStruct((8, F), jnp.float32),
        ],
    )(af, stats1, g1.reshape(1, F), b1.reshape(1, F), W1)


_SC_MESH = plsc.VectorSubcoreMesh(core_axis_name="c", subcore_axis_name="s")

_ZSTRIPE = ACC_R // NSUB
_HSTRIPE = HP // NSUB


@functools.partial(
    pl.kernel,
    mesh=_SC_MESH,
    compiler_params=pltpu.CompilerParams(use_tc_tiling_on_sc=False),
    out_type=[jax.ShapeDtypeStruct((ACC_R, CW), jnp.float32)] * 4,
    scratch_types=[
        pltpu.VMEM((IGRP, 128), jnp.int32),
        pltpu.VMEM((IGRP * 128,), jnp.int32),
        pltpu.VMEM((IGRP * 128,), jnp.int32),
        pltpu.VMEM((BS, CW), jnp.float32),
        pltpu.VMEM((BS, CW), jnp.float32),
        pltpu.VMEM((64, CW), jnp.float32),
        pltpu.VMEM_SHARED((ACC_R, CW), jnp.float32),
        pltpu.SemaphoreType.DMA,
        pltpu.SemaphoreType.DMA,
        pltpu.SemaphoreType.DMA,
        pltpu.SemaphoreType.DMA,
    ],
)
def _molconv_sc(h0, h1, h2, h3, pk_hbm, o0, o1, o2, o3,
                pk_v, beg_v, dst_v, rows_a, rows_b, zero_v, acc_sh,
                gsem_a, gsem_b, ssem_a, ssem_b):
    c = lax.axis_index("c")
    s = lax.axis_index("s")

    @pl.loop(0, 64)
    def _(i):
        @pl.loop(0, CW, step=16)
        def _(k):
            zero_v[i, pl.ds(k, 16)] = jnp.zeros((16,), jnp.float32)

    def chunk_pass(h_chunk_hbm, out_hbm):
        zbase = s * _ZSTRIPE
        for q in range(_ZSTRIPE // 64):
            pltpu.sync_copy(zero_v, acc_sh.at[pl.ds(zbase + q * 64, 64)])
        rem = _ZSTRIPE % 64
        if rem:
            pltpu.sync_copy(zero_v.at[pl.ds(0, rem)],
                            acc_sh.at[pl.ds(zbase + _ZSTRIPE - rem, rem)])
        plsc.subcore_barrier()

        def gat(i, buf, sem):
            return pltpu.async_copy(
                h_chunk_hbm.at[beg_v.at[pl.ds(i * BS, BS)]], buf, sem)

        def sca(i, buf, sem):
            return pltpu.async_copy(
                buf, acc_sh.at[dst_v.at[pl.ds(i * BS, BS)]], sem, add=True)

        @pl.loop(0, NGRP)
        def _(g):
            pltpu.sync_copy(pk_hbm.at[s, pl.ds(g * IGRP, IGRP)], pk_v)

            @pl.loop(0, IGRP)
            def _(r):
                @pl.loop(0, 128, step=16)
                def _(k):
                    w = pk_v[r, pl.ds(k, 16)]
                    beg_v[pl.ds(r * 128 + k, 16)] = w & ((1 << PKSHIFT) - 1)
                    dst_v[pl.ds(r * 128 + k, 16)] = lax.shift_right_logical(
                        w, PKSHIFT)

            ga = gat(0, rows_a, gsem_a)
            gb = gat(1, rows_b, gsem_b)
            ga.wait()
            sa = sca(0, rows_a, ssem_a)
            gb.wait()
            sb = sca(1, rows_b, ssem_b)
            for i in range(2, IGRP * 128 // BS, 2):
                sa.wait()
                ga = gat(i, rows_a, gsem_a)
                sb.wait()
                gb = gat(i + 1, rows_b, gsem_b)
                ga.wait()
                sa = sca(i, rows_a, ssem_a)
                gb.wait()
                sb = sca(i + 1, rows_b, ssem_b)
            sa.wait()
            sb.wait()

        plsc.subcore_barrier()
        pltpu.sync_copy(acc_sh.at[pl.ds(s * _ZSTRIPE, _ZSTRIPE)],
                        out_hbm.at[pl.ds(s * _ZSTRIPE, _ZSTRIPE)])
        plsc.subcore_barrier()

    for j in range(2):
        @pl.when(c == 0)
        def _(j=j):
            chunk_pass((h0, h1)[j], (o0, o1)[j])

        @pl.when(c == 1)
        def _(j=j):
            chunk_pass((h2, h3)[j], (o2, o3)[j])


def _panel_specs():
    return [pl.BlockSpec((RB, CW), (lambda i, t=t: (t * (PR // RB) + i, 0)))
            for t in range(NBT) for _ in range(4)]


def _stats2_body(*refs):
    panel_refs, o_ref = refs[:-1], refs[-1]
    s1, s2 = [], []
    for t in range(NBT):
        x = jnp.concatenate([panel_refs[4 * t + cc][...] for cc in range(4)],
                            axis=-1)
        s1.append(jnp.sum(x, axis=0, keepdims=True))
        s2.append(jnp.sum(x * x, axis=0, keepdims=True))
    upd = jnp.concatenate(
        [jnp.concatenate(s1, axis=-1),
         jnp.concatenate(s2, axis=-1),
         jnp.zeros((6, NBT * F), jnp.float32)], axis=0)

    @pl.when(pl.program_id(0) == 0)
    def _():
        o_ref[...] = jnp.zeros_like(o_ref)

    o_ref[...] += upd


def _stats2(bufs):
    return pl.pallas_call(
        _stats2_body,
        grid=(N // RB,),
        in_specs=_panel_specs(),
        out_specs=pl.BlockSpec((8, NBT * F), lambda i: (0, 0)),
        out_shape=jax.ShapeDtypeStruct((8, NBT * F), jnp.float32),
    )(*(bufs * NBT))


def _out_body(*refs):
    h_ref = refs[0]
    panel_refs = refs[1:17]
    hs_ref, bs_ref, g_ref, b_ref, w_ref, o_ref = refs[17:]
    a = _bn_elu(h_ref[...], hs_ref[0:1, :], hs_ref[1:2, :],
                g_ref[0:1, 0:F], b_ref[0:1, 0:F])
    acc = jnp.dot(a, w_ref[0:F, :], preferred_element_type=jnp.float32)
    for t in range(NBT):
        x = jnp.concatenate([panel_refs[4 * t + cc][...] for cc in range(4)],
                            axis=-1)
        c0 = (t + 1) * F
        at = _bn_elu(x, bs_ref[0:1, t * F:(t + 1) * F],
                     bs_ref[1:2, t * F:(t + 1) * F],
                     g_ref[0:1, c0:c0 + F], b_ref[0:1, c0:c0 + F])
        acc += jnp.dot(at, w_ref[c0:c0 + F, :],
                       preferred_element_type=jnp.float32)
    o_ref[...] = acc


def _head(h, bufs, hstats, bstats, g2, b2, W2):
    cd = (NBT + 1) * F
    return pl.pallas_call(
        _out_body,
        grid=(N // RB,),
        in_specs=[pl.BlockSpec((RB, F), lambda i: (i, 0))] + _panel_specs() + [
            pl.BlockSpec((8, F), lambda i: (0, 0)),
            pl.BlockSpec((8, NBT * F), lambda i: (0, 0)),
            pl.BlockSpec((1, cd), lambda i: (0, 0)),
            pl.BlockSpec((1, cd), lambda i: (0, 0)),
            pl.BlockSpec((cd, F), lambda i: (0, 0)),
        ],
        out_specs=pl.BlockSpec((RB, F), lambda i: (i, 0)),
        out_shape=jax.ShapeDtypeStruct((N, F), jnp.float32),
    )(h, *(bufs * NBT), hstats, bstats,
      g2.reshape(1, cd), b2.reshape(1, cd), W2)


def kernel(atom_features_list, bond_info, bn_gamma1, bn_beta1, W1,
           bn_gamma2, bn_beta2, W2):
    af = atom_features_list
    pk = _edge_indices(bond_info)
    stats1 = _stats1(af)
    h, h0, h1, h2, h3, hstats = _bottleneck(af, stats1, bn_gamma1, bn_beta1, W1)
    bufs = list(_molconv_sc(h0, h1, h2, h3, pk))
    bstats = _stats2(bufs)
    return _head(h, bufs, hstats, bstats, bn_gamma2, bn_beta2, W2)


# speedup vs baseline: 2.6815x; 1.1446x over previous
import functools

import jax
import jax.numpy as jnp
from jax import lax
from jax.experimental import pallas as pl
from jax.experimental.pallas import tpu as pltpu
from jax.experimental.pallas import tpu_sc as plsc

N = 10000
E = 320000
NBT = 4
F = 128
CW = 32
NSUB = 16
EPT = 20480
EPAD = NSUB * EPT
ROWS_B = EPT // 128
HP = 10240
PR = 10400
ACC_R = NBT * PR
DUMMY = N
EB = 2000
RB = 400
IGRP = 4
NGRP = ROWS_B // IGRP
BS = 128
_HSTRIPE = HP // NSUB


PKSHIFT = 14


def _edge_body(bond_ref, pk_ref):
    blk = bond_ref[...]
    dst = (blk[:, 2] & (NBT - 1)) * PR + blk[:, 1]
    pk_ref[0, 0, :] = dst * (1 << PKSHIFT) + blk[:, 0]


def _edge_indices(bond_info):
    pkf = pl.pallas_call(
        _edge_body,
        grid=(E // EB,),
        in_specs=[pl.BlockSpec((EB, 3), lambda i: (i, 0))],
        out_specs=pl.BlockSpec((1, 1, EB), lambda i: (i, 0, 0)),
        out_shape=jax.ShapeDtypeStruct((E // EB, 1, EB), jnp.int32),
    )(bond_info)
    pad = jnp.full((EPAD - E,), DUMMY * (1 << PKSHIFT), jnp.int32)
    pk = jnp.concatenate([pkf.reshape(E), pad])
    return pk.reshape(NSUB, ROWS_B, 128)


def _stats1_body(af_ref, o_ref):
    x = jnp.concatenate([af_ref[0], af_ref[1]], axis=-1)
    upd = jnp.concatenate(
        [jnp.sum(x, axis=0, keepdims=True),
         jnp.sum(x * x, axis=0, keepdims=True),
         jnp.zeros((6, F), jnp.float32)], axis=0)

    @pl.when(pl.program_id(0) == 0)
    def _():
        o_ref[...] = jnp.zeros_like(o_ref)

    o_ref[...] += upd


def _stats1(af):
    return pl.pallas_call(
        _stats1_body,
        grid=(N // RB,),
        in_specs=[pl.BlockSpec((2, RB, 64), lambda i: (0, i, 0))],
        out_specs=pl.BlockSpec((8, F), lambda i: (0, 0)),
        out_shape=jax.ShapeDtypeStruct((8, F), jnp.float32),
    )(af)


def _bn_elu(x, s1, s2, g, b):
    m = s1 / N
    v = s2 / N - m * m
    xn = (x - m) * lax.rsqrt(v + 1e-5) * g + b
    return jnp.where(xn > 0, xn, jnp.exp(xn) - 1.0)


def _h_body(af_ref, st_ref, g_ref, b_ref, w_ref,
            h_ref, c0_ref, c1_ref, c2_ref, c3_ref, hs_ref):
    x = jnp.concatenate([af_ref[0], af_ref[1]], axis=-1)
    a = _bn_elu(x, st_ref[0:1, :], st_ref[1:2, :], g_ref[...], b_ref[...])
    h = jnp.dot(a, w_ref[...], preferred_element_type=jnp.float32)
    h_ref[...] = h
    c0_ref[...] = h[:, 0 * CW:1 * CW]
    c1_ref[...] = h[:, 1 * CW:2 * CW]
    c2_ref[...] = h[:, 2 * CW:3 * CW]
    c3_ref[...] = h[:, 3 * CW:4 * CW]
    upd = jnp.concatenate(
        [jnp.sum(h, axis=0, keepdims=True),
         jnp.sum(h * h, axis=0, keepdims=True),
         jnp.zeros((6, F), jnp.float32)], axis=0)

    @pl.when(pl.program_id(0) == 0)
    def _():
        hs_ref[...] = jnp.zeros_like(hs_ref)

    hs_ref[...] += upd


def _bottleneck(af, stats1, g1, b1, W1):
    chunk_spec = pl.BlockSpec((RB, CW), lambda i: (i, 0))
    return pl.pallas_call(
        _h_body,
        grid=(N // RB,),
        in_specs=[
            pl.BlockSpec((2, RB, 64), lambda i: (0, i, 0)),
            pl.BlockSpec((8, F), lambda i: (0, 0)),
            pl.BlockSpec((1, F), lambda i: (0, 0)),
            pl.BlockSpec((1, F), lambda i: (0, 0)),
            pl.BlockSpec((F, F), lambda i: (0, 0)),
        ],
        out_specs=[
            pl.BlockSpec((RB, F), lambda i: (i, 0)),
            chunk_spec, chunk_spec, chunk_spec, chunk_spec,
            pl.BlockSpec((8, F), lambda i: (0, 0)),
        ],
        out_shape=[
            jax.ShapeDtypeStruct((HP, F), jnp.float32),
            jax.ShapeDtypeStruct((HP, CW), jnp.float32),
            jax.ShapeDtypeStruct((HP, CW), jnp.float32),
            jax.ShapeDtypeStruct((HP, CW), jnp.float32),
            jax.ShapeDtypeStruct((HP, CW), jnp.float32),
            jax.ShapeDtypeStruct((8, F), jnp.float32),
        ],
    )(af, stats1, g1.reshape(1, F), b1.reshape(1, F), W1)


_SC_MESH = plsc.VectorSubcoreMesh(core_axis_name="c", subcore_axis_name="s")

_ZSTRIPE = ACC_R // NSUB
_HSTRIPE = HP // NSUB


@functools.partial(
    pl.kernel,
    mesh=_SC_MESH,
    compiler_params=pltpu.CompilerParams(use_tc_tiling_on_sc=False),
    out_type=[jax.ShapeDtypeStruct((ACC_R, CW), jnp.float32)] * 4,
    scratch_types=[
        pltpu.VMEM((IGRP, 128), jnp.int32),
        pltpu.VMEM((IGRP * 128,), jnp.int32),
        pltpu.VMEM((IGRP * 128,), jnp.int32),
        pltpu.VMEM((BS, CW), jnp.float32),
        pltpu.VMEM((BS, CW), jnp.float32),
        pltpu.VMEM((32, CW), jnp.float32),
        pltpu.VMEM_SHARED((HP, CW), jnp.float32),
        pltpu.VMEM_SHARED((ACC_R, CW), jnp.float32),
        pltpu.SemaphoreType.DMA,
        pltpu.SemaphoreType.DMA,
        pltpu.SemaphoreType.DMA,
        pltpu.SemaphoreType.DMA,
    ],
)
def _molconv_sc(h0, h1, h2, h3, pk_hbm, o0, o1, o2, o3,
                pk_v, beg_v, dst_v, rows_a, rows_b, zero_v, h_sh, acc_sh,
                gsem_a, gsem_b, ssem_a, ssem_b):
    c = lax.axis_index("c")
    s = lax.axis_index("s")

    @pl.loop(0, 32)
    def _(i):
        @pl.loop(0, CW, step=16)
        def _(k):
            zero_v[i, pl.ds(k, 16)] = jnp.zeros((16,), jnp.float32)

    def chunk_pass(h_chunk_hbm, out_hbm):
        zbase = s * _ZSTRIPE
        for q in range(_ZSTRIPE // 32):
            pltpu.sync_copy(zero_v, acc_sh.at[pl.ds(zbase + q * 32, 32)])
        rem = _ZSTRIPE % 32
        if rem:
            pltpu.sync_copy(zero_v.at[pl.ds(0, rem)],
                            acc_sh.at[pl.ds(zbase + _ZSTRIPE - rem, rem)])
        pltpu.sync_copy(h_chunk_hbm.at[pl.ds(s * _HSTRIPE, _HSTRIPE)],
                        h_sh.at[pl.ds(s * _HSTRIPE, _HSTRIPE)])
        plsc.subcore_barrier()

        def gat(i, buf, sem):
            return pltpu.async_copy(
                h_sh.at[beg_v.at[pl.ds(i * BS, BS)]], buf, sem)

        def sca(i, buf, sem):
            return pltpu.async_copy(
                buf, acc_sh.at[dst_v.at[pl.ds(i * BS, BS)]], sem, add=True)

        @pl.loop(0, NGRP)
        def _(g):
            pltpu.sync_copy(pk_hbm.at[s, pl.ds(g * IGRP, IGRP)], pk_v)

            @pl.loop(0, IGRP)
            def _(r):
                @pl.loop(0, 128, step=16)
                def _(k):
                    w = pk_v[r, pl.ds(k, 16)]
                    beg_v[pl.ds(r * 128 + k, 16)] = w & ((1 << PKSHIFT) - 1)
                    dst_v[pl.ds(r * 128 + k, 16)] = lax.shift_right_logical(
                        w, PKSHIFT)

            ga = gat(0, rows_a, gsem_a)
            gb = gat(1, rows_b, gsem_b)
            ga.wait()
            sa = sca(0, rows_a, ssem_a)
            gb.wait()
            sb = sca(1, rows_b, ssem_b)
            for i in range(2, IGRP * 128 // BS, 2):
                sa.wait()
                ga = gat(i, rows_a, gsem_a)
                sb.wait()
                gb = gat(i + 1, rows_b, gsem_b)
                ga.wait()
                sa = sca(i, rows_a, ssem_a)
                gb.wait()
                sb = sca(i + 1, rows_b, ssem_b)
            sa.wait()
            sb.wait()

        plsc.subcore_barrier()
        pltpu.sync_copy(acc_sh.at[pl.ds(s * _ZSTRIPE, _ZSTRIPE)],
                        out_hbm.at[pl.ds(s * _ZSTRIPE, _ZSTRIPE)])
        plsc.subcore_barrier()

    for j in range(2):
        @pl.when(c == 0)
        def _(j=j):
            chunk_pass((h0, h1)[j], (o0, o1)[j])

        @pl.when(c == 1)
        def _(j=j):
            chunk_pass((h2, h3)[j], (o2, o3)[j])


def _panel_specs():
    return [pl.BlockSpec((RB, CW), (lambda i, t=t: (t * (PR // RB) + i, 0)))
            for t in range(NBT) for _ in range(4)]


def _stats2_body(*refs):
    panel_refs, o_ref = refs[:-1], refs[-1]
    s1, s2 = [], []
    for t in range(NBT):
        x = jnp.concatenate([panel_refs[4 * t + cc][...] for cc in range(4)],
                            axis=-1)
        s1.append(jnp.sum(x, axis=0, keepdims=True))
        s2.append(jnp.sum(x * x, axis=0, keepdims=True))
    upd = jnp.concatenate(
        [jnp.concatenate(s1, axis=-1),
         jnp.concatenate(s2, axis=-1),
         jnp.zeros((6, NBT * F), jnp.float32)], axis=0)

    @pl.when(pl.program_id(0) == 0)
    def _():
        o_ref[...] = jnp.zeros_like(o_ref)

    o_ref[...] += upd


def _stats2(bufs):
    return pl.pallas_call(
        _stats2_body,
        grid=(N // RB,),
        in_specs=_panel_specs(),
        out_specs=pl.BlockSpec((8, NBT * F), lambda i: (0, 0)),
        out_shape=jax.ShapeDtypeStruct((8, NBT * F), jnp.float32),
    )(*(bufs * NBT))


def _out_body(*refs):
    h_ref = refs[0]
    panel_refs = refs[1:17]
    hs_ref, bs_ref, g_ref, b_ref, w_ref, o_ref = refs[17:]
    a = _bn_elu(h_ref[...], hs_ref[0:1, :], hs_ref[1:2, :],
                g_ref[0:1, 0:F], b_ref[0:1, 0:F])
    acc = jnp.dot(a, w_ref[0:F, :], preferred_element_type=jnp.float32)
    for t in range(NBT):
        x = jnp.concatenate([panel_refs[4 * t + cc][...] for cc in range(4)],
                            axis=-1)
        c0 = (t + 1) * F
        at = _bn_elu(x, bs_ref[0:1, t * F:(t + 1) * F],
                     bs_ref[1:2, t * F:(t + 1) * F],
                     g_ref[0:1, c0:c0 + F], b_ref[0:1, c0:c0 + F])
        acc += jnp.dot(at, w_ref[c0:c0 + F, :],
                       preferred_element_type=jnp.float32)
    o_ref[...] = acc


def _head(h, bufs, hstats, bstats, g2, b2, W2):
    cd = (NBT + 1) * F
    return pl.pallas_call(
        _out_body,
        grid=(N // RB,),
        in_specs=[pl.BlockSpec((RB, F), lambda i: (i, 0))] + _panel_specs() + [
            pl.BlockSpec((8, F), lambda i: (0, 0)),
            pl.BlockSpec((8, NBT * F), lambda i: (0, 0)),
            pl.BlockSpec((1, cd), lambda i: (0, 0)),
            pl.BlockSpec((1, cd), lambda i: (0, 0)),
            pl.BlockSpec((cd, F), lambda i: (0, 0)),
        ],
        out_specs=pl.BlockSpec((RB, F), lambda i: (i, 0)),
        out_shape=jax.ShapeDtypeStruct((N, F), jnp.float32),
    )(h, *(bufs * NBT), hstats, bstats,
      g2.reshape(1, cd), b2.reshape(1, cd), W2)


def kernel(atom_features_list, bond_info, bn_gamma1, bn_beta1, W1,
           bn_gamma2, bn_beta2, W2):
    af = atom_features_list
    pk = _edge_indices(bond_info)
    stats1 = _stats1(af)
    h, h0, h1, h2, h3, hstats = _bottleneck(af, stats1, bn_gamma1, bn_beta1, W1)
    bufs = list(_molconv_sc(h0, h1, h2, h3, pk))
    bstats = _stats2(bufs)
    return _head(h, bufs, hstats, bstats, bn_gamma2, bn_beta2, W2)


# speedup vs baseline: 2.8926x; 1.0787x over previous
import functools

import jax
import jax.numpy as jnp
from jax import lax
from jax.experimental import pallas as pl
from jax.experimental.pallas import tpu as pltpu
from jax.experimental.pallas import tpu_sc as plsc

N = 10000
E = 320000
NBT = 4
F = 128
CW = 32
NSUB = 16
EPT = 20480
EPAD = NSUB * EPT
ROWS_B = EPT // 128
HP = 10240
PR = 10400
ACC_R = NBT * PR
DUMMY = N
EB = 2000
RB = 400
IGRP = 8
NGRP = ROWS_B // IGRP
BS = 128
_HSTRIPE = HP // NSUB


PKSHIFT = 14


def _edge_body(bond_ref, pk_ref):
    blk = bond_ref[...]
    dst = (blk[:, 2] & (NBT - 1)) * PR + blk[:, 1]
    pk_ref[0, 0, :] = dst * (1 << PKSHIFT) + blk[:, 0]


def _edge_indices(bond_info):
    pkf = pl.pallas_call(
        _edge_body,
        grid=(E // EB,),
        in_specs=[pl.BlockSpec((EB, 3), lambda i: (i, 0))],
        out_specs=pl.BlockSpec((1, 1, EB), lambda i: (i, 0, 0)),
        out_shape=jax.ShapeDtypeStruct((E // EB, 1, EB), jnp.int32),
    )(bond_info)
    pad = jnp.full((EPAD - E,), DUMMY * (1 << PKSHIFT), jnp.int32)
    pk = jnp.concatenate([pkf.reshape(E), pad])
    return pk.reshape(NSUB, ROWS_B, 128)


def _stats1_body(af_ref, o_ref):
    x = jnp.concatenate([af_ref[0], af_ref[1]], axis=-1)
    upd = jnp.concatenate(
        [jnp.sum(x, axis=0, keepdims=True),
         jnp.sum(x * x, axis=0, keepdims=True),
         jnp.zeros((6, F), jnp.float32)], axis=0)

    @pl.when(pl.program_id(0) == 0)
    def _():
        o_ref[...] = jnp.zeros_like(o_ref)

    o_ref[...] += upd


def _stats1(af):
    return pl.pallas_call(
        _stats1_body,
        grid=(N // RB,),
        in_specs=[pl.BlockSpec((2, RB, 64), lambda i: (0, i, 0))],
        out_specs=pl.BlockSpec((8, F), lambda i: (0, 0)),
        out_shape=jax.ShapeDtypeStruct((8, F), jnp.float32),
    )(af)


def _bn_elu(x, s1, s2, g, b):
    m = s1 / N
    v = s2 / N - m * m
    xn = (x - m) * lax.rsqrt(v + 1e-5) * g + b
    return jnp.where(xn > 0, xn, jnp.exp(xn) - 1.0)


def _h_body(af_ref, st_ref, g_ref, b_ref, w_ref,
            h_ref, c0_ref, c1_ref, c2_ref, c3_ref, hs_ref):
    x = jnp.concatenate([af_ref[0], af_ref[1]], axis=-1)
    a = _bn_elu(x, st_ref[0:1, :], st_ref[1:2, :], g_ref[...], b_ref[...])
    h = jnp.dot(a, w_ref[...], preferred_element_type=jnp.float32)
    h_ref[...] = h
    c0_ref[...] = h[:, 0 * CW:1 * CW]
    c1_ref[...] = h[:, 1 * CW:2 * CW]
    c2_ref[...] = h[:, 2 * CW:3 * CW]
    c3_ref[...] = h[:, 3 * CW:4 * CW]
    upd = jnp.concatenate(
        [jnp.sum(h, axis=0, keepdims=True),
         jnp.sum(h * h, axis=0, keepdims=True),
         jnp.zeros((6, F), jnp.float32)], axis=0)

    @pl.when(pl.program_id(0) == 0)
    def _():
        hs_ref[...] = jnp.zeros_like(hs_ref)

    hs_ref[...] += upd


def _bottleneck(af, stats1, g1, b1, W1):
    chunk_spec = pl.BlockSpec((RB, CW), lambda i: (i, 0))
    return pl.pallas_call(
        _h_body,
        grid=(N // RB,),
        in_specs=[
            pl.BlockSpec((2, RB, 64), lambda i: (0, i, 0)),
            pl.BlockSpec((8, F), lambda i: (0, 0)),
            pl.BlockSpec((1, F), lambda i: (0, 0)),
            pl.BlockSpec((1, F), lambda i: (0, 0)),
            pl.BlockSpec((F, F), lambda i: (0, 0)),
        ],
        out_specs=[
            pl.BlockSpec((RB, F), lambda i: (i, 0)),
            chunk_spec, chunk_spec, chunk_spec, chunk_spec,
            pl.BlockSpec((8, F), lambda i: (0, 0)),
        ],
        out_shape=[
            jax.ShapeDtypeStruct((HP, F), jnp.float32),
            jax.ShapeDtypeStruct((HP, CW), jnp.float32),
            jax.ShapeDtypeStruct((HP, CW), jnp.float32),
            jax.ShapeDtypeStruct((HP, CW), jnp.float32),
            jax.ShapeDtypeStruct((HP, CW), jnp.float32),
            jax.ShapeDtypeStruct((8, F), jnp.float32),
        ],
    )(af, stats1, g1.reshape(1, F), b1.reshape(1, F), W1)


_SC_MESH = plsc.VectorSubcoreMesh(core_axis_name="c", subcore_axis_name="s")

_ZSTRIPE = ACC_R // NSUB
_HSTRIPE = HP // NSUB


@functools.partial(
    pl.kernel,
    mesh=_SC_MESH,
    compiler_params=pltpu.CompilerParams(use_tc_tiling_on_sc=False),
    out_type=[jax.ShapeDtypeStruct((ACC_R, CW), jnp.float32)] * 4,
    scratch_types=[
        pltpu.VMEM((IGRP, 128), jnp.int32),
        pltpu.VMEM((IGRP * 128,), jnp.int32),
        pltpu.VMEM((IGRP * 128,), jnp.int32),
        pltpu.VMEM((BS, CW), jnp.float32),
        pltpu.VMEM((BS, CW), jnp.float32),
        pltpu.VMEM((BS, CW), jnp.float32),
        pltpu.VMEM((BS, CW), jnp.float32),
        pltpu.VMEM((32, CW), jnp.float32),
        pltpu.VMEM_SHARED((HP, CW), jnp.float32),
        pltpu.VMEM_SHARED((ACC_R, CW), jnp.float32),
        pltpu.SemaphoreType.DMA,
        pltpu.SemaphoreType.DMA,
        pltpu.SemaphoreType.DMA,
        pltpu.SemaphoreType.DMA,
        pltpu.SemaphoreType.DMA,
        pltpu.SemaphoreType.DMA,
        pltpu.SemaphoreType.DMA,
        pltpu.SemaphoreType.DMA,
    ],
)
def _molconv_sc(h0, h1, h2, h3, pk_hbm, o0, o1, o2, o3,
                pk_v, beg_v, dst_v, rows_a, rows_b, rows_c, rows_d,
                zero_v, h_sh, acc_sh,
                gsem_a, gsem_b, gsem_c, gsem_d,
                ssem_a, ssem_b, ssem_c, ssem_d):
    c = lax.axis_index("c")
    s = lax.axis_index("s")

    @pl.loop(0, 32)
    def _(i):
        @pl.loop(0, CW, step=16)
        def _(k):
            zero_v[i, pl.ds(k, 16)] = jnp.zeros((16,), jnp.float32)

    def chunk_pass(h_chunk_hbm, out_hbm):
        zbase = s * _ZSTRIPE
        for q in range(_ZSTRIPE // 32):
            pltpu.sync_copy(zero_v, acc_sh.at[pl.ds(zbase + q * 32, 32)])
        rem = _ZSTRIPE % 32
        if rem:
            pltpu.sync_copy(zero_v.at[pl.ds(0, rem)],
                            acc_sh.at[pl.ds(zbase + _ZSTRIPE - rem, rem)])
        pltpu.sync_copy(h_chunk_hbm.at[pl.ds(s * _HSTRIPE, _HSTRIPE)],
                        h_sh.at[pl.ds(s * _HSTRIPE, _HSTRIPE)])
        plsc.subcore_barrier()

        def gat(i, buf, sem):
            return pltpu.async_copy(
                h_sh.at[beg_v.at[pl.ds(i * BS, BS)]], buf, sem)

        def sca(i, buf, sem):
            return pltpu.async_copy(
                buf, acc_sh.at[dst_v.at[pl.ds(i * BS, BS)]], sem, add=True)

        @pl.loop(0, NGRP)
        def _(g):
            pltpu.sync_copy(pk_hbm.at[s, pl.ds(g * IGRP, IGRP)], pk_v)

            @pl.loop(0, IGRP)
            def _(r):
                @pl.loop(0, 128, step=16)
                def _(k):
                    w = pk_v[r, pl.ds(k, 16)]
                    beg_v[pl.ds(r * 128 + k, 16)] = w & ((1 << PKSHIFT) - 1)
                    dst_v[pl.ds(r * 128 + k, 16)] = lax.shift_right_logical(
                        w, PKSHIFT)

            bufs = (rows_a, rows_b, rows_c, rows_d)
            gsems = (gsem_a, gsem_b, gsem_c, gsem_d)
            ssems = (ssem_a, ssem_b, ssem_c, ssem_d)
            nops = IGRP * 128 // BS
            gh = [gat(i, bufs[i], gsems[i]) for i in range(4)]
            sh = [None] * 4
            for base in range(0, nops, 4):
                for i in range(4):
                    gh[i].wait()
                    sh[i] = sca(base + i, bufs[i], ssems[i])
                if base + 4 < nops:
                    for i in range(4):
                        sh[i].wait()
                        gh[i] = gat(base + 4 + i, bufs[i], gsems[i])
            for i in range(4):
                sh[i].wait()

        plsc.subcore_barrier()
        pltpu.sync_copy(acc_sh.at[pl.ds(s * _ZSTRIPE, _ZSTRIPE)],
                        out_hbm.at[pl.ds(s * _ZSTRIPE, _ZSTRIPE)])
        plsc.subcore_barrier()

    for j in range(2):
        @pl.when(c == 0)
        def _(j=j):
            chunk_pass((h0, h1)[j], (o0, o1)[j])

        @pl.when(c == 1)
        def _(j=j):
            chunk_pass((h2, h3)[j], (o2, o3)[j])


def _panel_specs():
    return [pl.BlockSpec((RB, CW), (lambda i, t=t: (t * (PR // RB) + i, 0)))
            for t in range(NBT) for _ in range(4)]


def _stats2_body(*refs):
    panel_refs, o_ref = refs[:-1], refs[-1]
    s1, s2 = [], []
    for t in range(NBT):
        x = jnp.concatenate([panel_refs[4 * t + cc][...] for cc in range(4)],
                            axis=-1)
        s1.append(jnp.sum(x, axis=0, keepdims=True))
        s2.append(jnp.sum(x * x, axis=0, keepdims=True))
    upd = jnp.concatenate(
        [jnp.concatenate(s1, axis=-1),
         jnp.concatenate(s2, axis=-1),
         jnp.zeros((6, NBT * F), jnp.float32)], axis=0)

    @pl.when(pl.program_id(0) == 0)
    def _():
        o_ref[...] = jnp.zeros_like(o_ref)

    o_ref[...] += upd


def _stats2(bufs):
    return pl.pallas_call(
        _stats2_body,
        grid=(N // RB,),
        in_specs=_panel_specs(),
        out_specs=pl.BlockSpec((8, NBT * F), lambda i: (0, 0)),
        out_shape=jax.ShapeDtypeStruct((8, NBT * F), jnp.float32),
    )(*(bufs * NBT))


def _out_body(*refs):
    h_ref = refs[0]
    panel_refs = refs[1:17]
    hs_ref, bs_ref, g_ref, b_ref, w_ref, o_ref = refs[17:]
    a = _bn_elu(h_ref[...], hs_ref[0:1, :], hs_ref[1:2, :],
                g_ref[0:1, 0:F], b_ref[0:1, 0:F])
    acc = jnp.dot(a, w_ref[0:F, :], preferred_element_type=jnp.float32)
    for t in range(NBT):
        x = jnp.concatenate([panel_refs[4 * t + cc][...] for cc in range(4)],
                            axis=-1)
        c0 = (t + 1) * F
        at = _bn_elu(x, bs_ref[0:1, t * F:(t + 1) * F],
                     bs_ref[1:2, t * F:(t + 1) * F],
                     g_ref[0:1, c0:c0 + F], b_ref[0:1, c0:c0 + F])
        acc += jnp.dot(at, w_ref[c0:c0 + F, :],
                       preferred_element_type=jnp.float32)
    o_ref[...] = acc


def _head(h, bufs, hstats, bstats, g2, b2, W2):
    cd = (NBT + 1) * F
    return pl.pallas_call(
        _out_body,
        grid=(N // RB,),
        in_specs=[pl.BlockSpec((RB, F), lambda i: (i, 0))] + _panel_specs() + [
            pl.BlockSpec((8, F), lambda i: (0, 0)),
            pl.BlockSpec((8, NBT * F), lambda i: (0, 0)),
            pl.BlockSpec((1, cd), lambda i: (0, 0)),
            pl.BlockSpec((1, cd), lambda i: (0, 0)),
            pl.BlockSpec((cd, F), lambda i: (0, 0)),
        ],
        out_specs=pl.BlockSpec((RB, F), lambda i: (i, 0)),
        out_shape=jax.ShapeDtypeStruct((N, F), jnp.float32),
    )(h, *(bufs * NBT), hstats, bstats,
      g2.reshape(1, cd), b2.reshape(1, cd), W2)


def kernel(atom_features_list, bond_info, bn_gamma1, bn_beta1, W1,
           bn_gamma2, bn_beta2, W2):
    af = atom_features_list
    pk = _edge_indices(bond_info)
    stats1 = _stats1(af)
    h, h0, h1, h2, h3, hstats = _bottleneck(af, stats1, bn_gamma1, bn_beta1, W1)
    bufs = list(_molconv_sc(h0, h1, h2, h3, pk))
    bstats = _stats2(bufs)
    return _head(h, bufs, hstats, bstats, bn_gamma2, bn_beta2, W2)


# speedup vs baseline: 3.5825x; 1.2385x over previous
import functools

import jax
import jax.numpy as jnp
from jax import lax
from jax.experimental import pallas as pl
from jax.experimental.pallas import tpu as pltpu
from jax.experimental.pallas import tpu_sc as plsc

N = 10000
E = 320000
NBT = 4
F = 128
CW = 64
NSUB = 16
EPT = 20480
EPAD = NSUB * EPT
ROWS_B = EPT // 128
HP = 10240
PR = 10400
ACC_R = NBT * PR
DUMMY = N
EB = 2000
RB = 400
IGRP = 8
NGRP = ROWS_B // IGRP
BS = 128
_HSTRIPE = HP // NSUB


PKSHIFT = 14


def _edge_body(bond_ref, pk_ref):
    blk = bond_ref[...]
    dst = (blk[:, 2] & (NBT - 1)) * PR + blk[:, 1]
    pk_ref[0, 0, :] = dst * (1 << PKSHIFT) + blk[:, 0]


def _edge_indices(bond_info):
    pkf = pl.pallas_call(
        _edge_body,
        grid=(E // EB,),
        in_specs=[pl.BlockSpec((EB, 3), lambda i: (i, 0))],
        out_specs=pl.BlockSpec((1, 1, EB), lambda i: (i, 0, 0)),
        out_shape=jax.ShapeDtypeStruct((E // EB, 1, EB), jnp.int32),
    )(bond_info)
    pad = jnp.full((EPAD - E,), DUMMY * (1 << PKSHIFT), jnp.int32)
    pk = jnp.concatenate([pkf.reshape(E), pad])
    return pk.reshape(NSUB, ROWS_B, 128)


def _stats1_body(af_ref, o_ref):
    x = jnp.concatenate([af_ref[0], af_ref[1]], axis=-1)
    upd = jnp.concatenate(
        [jnp.sum(x, axis=0, keepdims=True),
         jnp.sum(x * x, axis=0, keepdims=True),
         jnp.zeros((6, F), jnp.float32)], axis=0)

    @pl.when(pl.program_id(0) == 0)
    def _():
        o_ref[...] = jnp.zeros_like(o_ref)

    o_ref[...] += upd


def _stats1(af):
    return pl.pallas_call(
        _stats1_body,
        grid=(N // RB,),
        in_specs=[pl.BlockSpec((2, RB, 64), lambda i: (0, i, 0))],
        out_specs=pl.BlockSpec((8, F), lambda i: (0, 0)),
        out_shape=jax.ShapeDtypeStruct((8, F), jnp.float32),
    )(af)


def _bn_elu(x, s1, s2, g, b):
    m = s1 / N
    v = s2 / N - m * m
    xn = (x - m) * lax.rsqrt(v + 1e-5) * g + b
    return jnp.where(xn > 0, xn, jnp.exp(xn) - 1.0)


def _h_body(af_ref, st_ref, g_ref, b_ref, w_ref,
            h_ref, c0_ref, c1_ref, hs_ref):
    x = jnp.concatenate([af_ref[0], af_ref[1]], axis=-1)
    a = _bn_elu(x, st_ref[0:1, :], st_ref[1:2, :], g_ref[...], b_ref[...])
    h = jnp.dot(a, w_ref[...], preferred_element_type=jnp.float32)
    h_ref[...] = h
    c0_ref[...] = h[:, 0 * CW:1 * CW].astype(jnp.bfloat16)
    c1_ref[...] = h[:, 1 * CW:2 * CW].astype(jnp.bfloat16)
    upd = jnp.concatenate(
        [jnp.sum(h, axis=0, keepdims=True),
         jnp.sum(h * h, axis=0, keepdims=True),
         jnp.zeros((6, F), jnp.float32)], axis=0)

    @pl.when(pl.program_id(0) == 0)
    def _():
        hs_ref[...] = jnp.zeros_like(hs_ref)

    hs_ref[...] += upd


def _bottleneck(af, stats1, g1, b1, W1):
    chunk_spec = pl.BlockSpec((RB, CW), lambda i: (i, 0))
    return pl.pallas_call(
        _h_body,
        grid=(N // RB,),
        in_specs=[
            pl.BlockSpec((2, RB, 64), lambda i: (0, i, 0)),
            pl.BlockSpec((8, F), lambda i: (0, 0)),
            pl.BlockSpec((1, F), lambda i: (0, 0)),
            pl.BlockSpec((1, F), lambda i: (0, 0)),
            pl.BlockSpec((F, F), lambda i: (0, 0)),
        ],
        out_specs=[
            pl.BlockSpec((RB, F), lambda i: (i, 0)),
            chunk_spec, chunk_spec,
            pl.BlockSpec((8, F), lambda i: (0, 0)),
        ],
        out_shape=[
            jax.ShapeDtypeStruct((HP, F), jnp.float32),
            jax.ShapeDtypeStruct((HP, CW), jnp.bfloat16),
            jax.ShapeDtypeStruct((HP, CW), jnp.bfloat16),
            jax.ShapeDtypeStruct((8, F), jnp.float32),
        ],
    )(af, stats1, g1.reshape(1, F), b1.reshape(1, F), W1)


_SC_MESH = plsc.VectorSubcoreMesh(core_axis_name="c", subcore_axis_name="s")

_ZSTRIPE = ACC_R // NSUB
_HSTRIPE = HP // NSUB


@functools.partial(
    pl.kernel,
    mesh=_SC_MESH,
    compiler_params=pltpu.CompilerParams(use_tc_tiling_on_sc=False),
    out_type=[jax.ShapeDtypeStruct((ACC_R, CW), jnp.bfloat16)] * 2,
    scratch_types=[
        pltpu.VMEM((IGRP, 128), jnp.int32),
        pltpu.VMEM((IGRP * 128,), jnp.int32),
        pltpu.VMEM((IGRP * 128,), jnp.int32),
        pltpu.VMEM((BS, CW), jnp.bfloat16),
        pltpu.VMEM((BS, CW), jnp.bfloat16),
        pltpu.VMEM((BS, CW), jnp.bfloat16),
        pltpu.VMEM((BS, CW), jnp.bfloat16),
        pltpu.VMEM((32, CW), jnp.bfloat16),
        pltpu.VMEM_SHARED((HP, CW), jnp.bfloat16),
        pltpu.VMEM_SHARED((ACC_R, CW), jnp.bfloat16),
        pltpu.SemaphoreType.DMA,
        pltpu.SemaphoreType.DMA,
        pltpu.SemaphoreType.DMA,
        pltpu.SemaphoreType.DMA,
        pltpu.SemaphoreType.DMA,
        pltpu.SemaphoreType.DMA,
        pltpu.SemaphoreType.DMA,
        pltpu.SemaphoreType.DMA,
    ],
)
def _molconv_sc(h0, h1, pk_hbm, o0, o1,
                pk_v, beg_v, dst_v, rows_a, rows_b, rows_c, rows_d,
                zero_v, h_sh, acc_sh,
                gsem_a, gsem_b, gsem_c, gsem_d,
                ssem_a, ssem_b, ssem_c, ssem_d):
    c = lax.axis_index("c")
    s = lax.axis_index("s")

    @pl.loop(0, 32)
    def _(i):
        @pl.loop(0, CW, step=32)
        def _(k):
            zero_v[i, pl.ds(k, 32)] = jnp.zeros((32,), jnp.bfloat16)

    def chunk_pass(h_chunk_hbm, out_hbm):
        zbase = s * _ZSTRIPE
        for q in range(_ZSTRIPE // 32):
            pltpu.sync_copy(zero_v, acc_sh.at[pl.ds(zbase + q * 32, 32)])
        rem = _ZSTRIPE % 32
        if rem:
            pltpu.sync_copy(zero_v.at[pl.ds(0, rem)],
                            acc_sh.at[pl.ds(zbase + _ZSTRIPE - rem, rem)])
        pltpu.sync_copy(h_chunk_hbm.at[pl.ds(s * _HSTRIPE, _HSTRIPE)],
                        h_sh.at[pl.ds(s * _HSTRIPE, _HSTRIPE)])
        plsc.subcore_barrier()

        def gat(i, buf, sem):
            return pltpu.async_copy(
                h_sh.at[beg_v.at[pl.ds(i * BS, BS)]], buf, sem)

        def sca(i, buf, sem):
            return pltpu.async_copy(
                buf, acc_sh.at[dst_v.at[pl.ds(i * BS, BS)]], sem, add=True)

        @pl.loop(0, NGRP)
        def _(g):
            pltpu.sync_copy(pk_hbm.at[s, pl.ds(g * IGRP, IGRP)], pk_v)

            @pl.loop(0, IGRP)
            def _(r):
                @pl.loop(0, 128, step=16)
                def _(k):
                    w = pk_v[r, pl.ds(k, 16)]
                    beg_v[pl.ds(r * 128 + k, 16)] = w & ((1 << PKSHIFT) - 1)
                    dst_v[pl.ds(r * 128 + k, 16)] = lax.shift_right_logical(
                        w, PKSHIFT)

            bufs = (rows_a, rows_b, rows_c, rows_d)
            gsems = (gsem_a, gsem_b, gsem_c, gsem_d)
            ssems = (ssem_a, ssem_b, ssem_c, ssem_d)
            nops = IGRP * 128 // BS
            gh = [gat(i, bufs[i], gsems[i]) for i in range(4)]
            sh = [None] * 4
            for base in range(0, nops, 4):
                for i in range(4):
                    gh[i].wait()
                    sh[i] = sca(base + i, bufs[i], ssems[i])
                if base + 4 < nops:
                    for i in range(4):
                        sh[i].wait()
                        gh[i] = gat(base + 4 + i, bufs[i], gsems[i])
            for i in range(4):
                sh[i].wait()

        plsc.subcore_barrier()
        pltpu.sync_copy(acc_sh.at[pl.ds(s * _ZSTRIPE, _ZSTRIPE)],
                        out_hbm.at[pl.ds(s * _ZSTRIPE, _ZSTRIPE)])
        plsc.subcore_barrier()

    @pl.when(c == 0)
    def _():
        chunk_pass(h0, o0)

    @pl.when(c == 1)
    def _():
        chunk_pass(h1, o1)


def _panel_specs():
    return [pl.BlockSpec((RB, CW), (lambda i, t=t: (t * (PR // RB) + i, 0)))
            for t in range(NBT) for _ in range(2)]


def _stats2_body(*refs):
    panel_refs, o_ref = refs[:-1], refs[-1]
    s1, s2 = [], []
    for t in range(NBT):
        x = jnp.concatenate([panel_refs[2 * t + cc][...] for cc in range(2)],
                            axis=-1).astype(jnp.float32)
        s1.append(jnp.sum(x, axis=0, keepdims=True))
        s2.append(jnp.sum(x * x, axis=0, keepdims=True))
    upd = jnp.concatenate(
        [jnp.concatenate(s1, axis=-1),
         jnp.concatenate(s2, axis=-1),
         jnp.zeros((6, NBT * F), jnp.float32)], axis=0)

    @pl.when(pl.program_id(0) == 0)
    def _():
        o_ref[...] = jnp.zeros_like(o_ref)

    o_ref[...] += upd


def _stats2(bufs):
    return pl.pallas_call(
        _stats2_body,
        grid=(N // RB,),
        in_specs=_panel_specs(),
        out_specs=pl.BlockSpec((8, NBT * F), lambda i: (0, 0)),
        out_shape=jax.ShapeDtypeStruct((8, NBT * F), jnp.float32),
    )(*(bufs * NBT))


def _out_body(*refs):
    h_ref = refs[0]
    panel_refs = refs[1:9]
    hs_ref, bs_ref, g_ref, b_ref, w_ref, o_ref = refs[9:]
    a = _bn_elu(h_ref[...], hs_ref[0:1, :], hs_ref[1:2, :],
                g_ref[0:1, 0:F], b_ref[0:1, 0:F])
    acc = jnp.dot(a, w_ref[0:F, :], preferred_element_type=jnp.float32)
    for t in range(NBT):
        x = jnp.concatenate([panel_refs[2 * t + cc][...] for cc in range(2)],
                            axis=-1).astype(jnp.float32)
        c0 = (t + 1) * F
        at = _bn_elu(x, bs_ref[0:1, t * F:(t + 1) * F],
                     bs_ref[1:2, t * F:(t + 1) * F],
                     g_ref[0:1, c0:c0 + F], b_ref[0:1, c0:c0 + F])
        acc += jnp.dot(at, w_ref[c0:c0 + F, :],
                       preferred_element_type=jnp.float32)
    o_ref[...] = acc


def _head(h, bufs, hstats, bstats, g2, b2, W2):
    cd = (NBT + 1) * F
    return pl.pallas_call(
        _out_body,
        grid=(N // RB,),
        in_specs=[pl.BlockSpec((RB, F), lambda i: (i, 0))] + _panel_specs() + [
            pl.BlockSpec((8, F), lambda i: (0, 0)),
            pl.BlockSpec((8, NBT * F), lambda i: (0, 0)),
            pl.BlockSpec((1, cd), lambda i: (0, 0)),
            pl.BlockSpec((1, cd), lambda i: (0, 0)),
            pl.BlockSpec((cd, F), lambda i: (0, 0)),
        ],
        out_specs=pl.BlockSpec((RB, F), lambda i: (i, 0)),
        out_shape=jax.ShapeDtypeStruct((N, F), jnp.float32),
    )(h, *(bufs * NBT), hstats, bstats,
      g2.reshape(1, cd), b2.reshape(1, cd), W2)


def kernel(atom_features_list, bond_info, bn_gamma1, bn_beta1, W1,
           bn_gamma2, bn_beta2, W2):
    af = atom_features_list
    pk = _edge_indices(bond_info)
    stats1 = _stats1(af)
    h, h0, h1, hstats = _bottleneck(af, stats1, bn_gamma1, bn_beta1, W1)
    bufs = list(_molconv_sc(h0, h1, pk))
    bstats = _stats2(bufs)
    return _head(h, bufs, hstats, bstats, bn_gamma2, bn_beta2, W2)
